# baseline (speedup 1.0000x reference)
import functools

import jax
import jax.numpy as jnp
from jax import lax
from jax.experimental import pallas as pl
from jax.experimental.pallas import tpu as pltpu
from jax.experimental.pallas import tpu_sc as plsc

_N = 10000
_E = 320000
_D = 128
_K = 128
_CHUNKS = _E // _K
_NC, _NS = 2, 16
_NW = _NC * _NS
_FULL = _CHUNKS // _NW
_EXTRA = _CHUNKS % _NW
_NP = 10240
_RPT = _NP // _NS

_mesh = plsc.VectorSubcoreMesh(core_axis_name="c", subcore_axis_name="s")


_GDN = lax.GatherDimensionNumbers(
    offset_dims=(), collapsed_slice_dims=(0,), start_index_map=(0,))


def _bcast_lane(vec, l):
    idx = jnp.full((16, 1), l, jnp.int32)
    return lax.gather(vec, idx, _GDN, (1,),
                      mode=lax.GatherScatterMode.PROMISE_IN_BOUNDS)


def _scale_rows(rows_v, pk_v):
    def scale_group(g, c2):
        wgrp = plsc.bitcast(pk_v[2, pl.ds(g * 16, 16)], jnp.float32)
        for l in range(16):
            wv = _bcast_lane(wgrp, l)
            e = g * 16 + l
            for q in range(_D // 16):
                sl = pl.ds(q * 16, 16)
                rows_v[e, sl] = rows_v[e, sl] * wv
        return c2

    lax.fori_loop(0, _K // 16, scale_group, 0)


def _sc_scatter_body(p_hbm, pk_hbm, z_hbm, out_hbm, *bufs):
    (pk0, pk1, pk2, rows0, rows1,
     i0, i1, i2, g0, g1, s0, s1, acc_sh) = bufs
    PK = [(pk0, i0), (pk1, i1), (pk2, i2)]
    RW = [(rows0, g0, s0), (rows1, g1, s1)]
    cid = lax.axis_index("c")
    sid = lax.axis_index("s")
    wid = sid * _NC + cid
    pltpu.sync_copy(z_hbm, acc_sh.at[pl.ds(sid * _RPT, _RPT)])
    plsc.subcore_barrier()

    def cix(j):
        return wid + j * _NW

    def issue_idx(c, k):
        pk_v, sem = PK[k]
        pltpu.async_copy(pk_hbm.at[c], pk_v, sem)

    def wait_idx(k):
        pk_v, sem = PK[k]
        pltpu.make_async_copy(pk_hbm.at[0], pk_v, sem).wait()

    def issue_gather(k, r):
        pltpu.async_copy(p_hbm.at[PK[k][0].at[0]], RW[r][0], RW[r][1])

    def wait_gather(k, r):
        pltpu.make_async_copy(p_hbm.at[PK[k][0].at[0]], RW[r][0],
                              RW[r][1]).wait()

    def issue_scatter(k, r):
        pltpu.async_copy(RW[r][0], acc_sh.at[PK[k][0].at[1]], RW[r][2],
                         add=True)

    def wait_scatter(k, r):
        pltpu.make_async_copy(RW[r][0], acc_sh.at[PK[k][0].at[1]],
                              RW[r][2]).wait()

    issue_idx(cix(0), 0)
    issue_idx(cix(1), 1)
    wait_idx(0)
    issue_gather(0, 0)

    last = _FULL - 1

    def step(t, off, j):
        wait_gather(off % 3, off % 2)

        if off == 0:
            @pl.when(t > 0)
            def _():
                wait_scatter((off + 2) % 3, (off + 1) % 2)
        else:
            wait_scatter((off + 2) % 3, (off + 1) % 2)

        def launch_next():
            wait_idx((off + 1) % 3)
            issue_gather((off + 1) % 3, (off + 1) % 2)

        def prefetch_idx():
            issue_idx(cix(j + 2), (off + 2) % 3)

        if off <= 3:
            launch_next()
            prefetch_idx()
        else:
            @pl.when(j < last)
            def _():
                launch_next()

            @pl.when(j + 2 <= last)
            def _():
                prefetch_idx()

        _scale_rows(RW[off % 2][0], PK[off % 3][0])
        issue_scatter(off % 3, off % 2)

    def six(t, carry):
        for off in range(6):
            step(t, off, 6 * t + off)
        return carry

    lax.fori_loop(0, _FULL // 6, six, 0)
    wait_scatter(2, 1)

    @pl.when(wid < _EXTRA)
    def _():
        c = _NW * _FULL + wid
        issue_idx(c, 0)
        wait_idx(0)
        issue_gather(0, 0)
        wait_gather(0, 0)
        _scale_rows(rows0, pk0)
        issue_scatter(0, 0)
        wait_scatter(0, 0)

    plsc.subcore_barrier()
    pltpu.sync_copy(acc_sh.at[pl.ds(sid * _RPT, _RPT)],
                    out_hbm.at[cid, pl.ds(sid * _RPT, _RPT)])


def _sc_bufs():
    return [
        pltpu.VMEM((3, _K), jnp.int32),
        pltpu.VMEM((3, _K), jnp.int32),
        pltpu.VMEM((3, _K), jnp.int32),
        pltpu.VMEM((_K, _D), jnp.float32),
        pltpu.VMEM((_K, _D), jnp.float32),
    ] + [pltpu.SemaphoreType.DMA] * 7


_sc_scatter = pl.kernel(
    _sc_scatter_body,
    out_type=jax.ShapeDtypeStruct((_NC, _NP, _D), jnp.float32),
    mesh=_mesh,
    scratch_types=_sc_bufs() + [pltpu.VMEM_SHARED((_NP, _D), jnp.float32)],
    compiler_params=pltpu.CompilerParams(needs_layout_passes=False),
)


def _sc_den_body(pk_hbm, out_hbm, *bufs):
    (pk0, pk1, den_v, pb, bc, i0, i1, part_sh) = bufs
    PK = [(pk0, i0), (pk1, i1)]
    cid = lax.axis_index("c")
    sid = lax.axis_index("s")
    wid = sid * _NC + cid

    def z(i, c):
        den_v[pl.ds(i * 16, 16)] = jnp.zeros((16,), jnp.float32)
        return c

    lax.fori_loop(0, _NP // 16, z, 0, unroll=8)

    def cix(j):
        return wid + j * _NW

    def issue_idx(c, k):
        pk_v, sem = PK[k]
        pltpu.async_copy(pk_hbm.at[c], pk_v, sem)

    def wait_idx(k):
        pk_v, sem = PK[k]
        pltpu.make_async_copy(pk_hbm.at[0], pk_v, sem).wait()

    def accum(k):
        pk_v, _ = PK[k]
        for g in range(_K // 16):
            sl = pl.ds(g * 16, 16)
            ii = pk_v[1, sl]
            vv = plsc.bitcast(pk_v[2, sl], jnp.float32)
            plsc.addupdate_scatter(den_v, [ii], vv)

    issue_idx(cix(0), 0)
    issue_idx(cix(1), 1)

    def two(t, carry):
        for off in range(2):
            j = 2 * t + off
            wait_idx(off)
            accum(off)

            @pl.when(j + 2 <= _FULL - 1)
            def _():
                issue_idx(cix(j + 2), off)
        return carry

    lax.fori_loop(0, _FULL // 2, two, 0)

    @pl.when(wid < _EXTRA)
    def _():
        issue_idx(_NW * _FULL + wid, 0)
        wait_idx(0)
        accum(0)

    pltpu.sync_copy(den_v, part_sh.at[sid])
    plsc.subcore_barrier()
    pltpu.sync_copy(part_sh.at[:, pl.ds(sid * _RPT, _RPT)], pb)

    def red(g, c):
        acc = pb[0, pl.ds(g * 16, 16)]
        for r in range(1, _NS):
            acc = acc + pb[r, pl.ds(g * 16, 16)]
        den_v[pl.ds(g * 16, 16)] = acc
        return c

    lax.fori_loop(0, _RPT // 16, red, 0, unroll=2)

    for b5 in range(_RPT // _K):
        def bcast(rr, c):
            wv = plsc.load_gather(
                den_v, [jnp.zeros((16,), jnp.int32) + (b5 * _K + rr)])
            for q in range(_D // 16):
                bc[rr, pl.ds(q * 16, 16)] = wv
            return c

        lax.fori_loop(0, _K, bcast, 0, unroll=4)
        pltpu.sync_copy(
            bc, out_hbm.at[cid, pl.ds(sid * _RPT + b5 * _K, _K)])


_sc_den = pl.kernel(
    _sc_den_body,
    out_type=jax.ShapeDtypeStruct((_NC, _NP, _D), jnp.float32),
    mesh=_mesh,
    scratch_types=[
        pltpu.VMEM((3, _K), jnp.int32),
        pltpu.VMEM((3, _K), jnp.int32),
        pltpu.VMEM((_NP,), jnp.float32),
        pltpu.VMEM((_NS, _RPT), jnp.float32),
        pltpu.VMEM((_K, _D), jnp.float32),
        pltpu.SemaphoreType.DMA,
        pltpu.SemaphoreType.DMA,
        pltpu.VMEM_SHARED((_NS, _NP), jnp.float32),
    ],
    compiler_params=pltpu.CompilerParams(needs_layout_passes=False),
)


_R = 1000


def _vec(b):
    return pl.BlockSpec((1, _D), lambda i: (0, 0))


def _ln(x, g, b):
    m = jnp.mean(x, axis=-1, keepdims=True)
    v = jnp.mean((x - m) * (x - m), axis=-1, keepdims=True)
    return (x - m) / jnp.sqrt(v + 1e-5) * g + b


def _in_body(x_ref, wi_ref, b_ref, wn_ref, den_ref, h_ref, p_ref, di_ref):
    h = (jnp.dot(x_ref[...], wi_ref[...],
                 preferred_element_type=jnp.float32) + b_ref[...])
    h_ref[...] = h
    p_ref[...] = jnp.dot(h, wn_ref[...], preferred_element_type=jnp.float32)
    den = den_ref[0, :, 0:1] + den_ref[1, :, 0:1]
    di_ref[...] = 1.0 / jnp.maximum(den, 1e-6)


def _in_fused(x, wi, b, wn, den2):
    blk = _NP // (_N // _R)
    return pl.pallas_call(
        _in_body,
        grid=(_N // _R,),
        in_specs=[
            pl.BlockSpec((_R, _D), lambda i: (i, 0)),
            pl.BlockSpec((_D, _D), lambda i: (0, 0)),
            _vec(b),
            pl.BlockSpec((_D, _D), lambda i: (0, 0)),
            pl.BlockSpec((_NC, blk, _D), lambda i: (0, i, 0)),
        ],
        out_specs=[pl.BlockSpec((_R, _D), lambda i: (i, 0))] * 2
        + [pl.BlockSpec((blk, 1), lambda i: (i, 0))],
        out_shape=[jax.ShapeDtypeStruct((_N, _D), jnp.float32)] * 2
        + [jax.ShapeDtypeStruct((_NP, 1), jnp.float32)],
    )(x, wi, b.reshape(1, _D), wn, den2)


def _post_core(h_ref, s_ref, di_ref, ws_ref, b_ref, g_ref, lb_ref):
    h = h_ref[...]
    s = s_ref[0] + s_ref[1]
    hn = (jnp.dot(h, ws_ref[...], preferred_element_type=jnp.float32)
          + s * di_ref[...] + b_ref[...])
    hn = jnp.maximum(hn, 0.0)
    hn = _ln(hn, g_ref[...], lb_ref[...])
    return h + hn


def _post_body(h_ref, s_ref, di_ref, ws_ref, b_ref, g_ref, lb_ref, wn_ref,
               ho_ref, p_ref):
    ho = _post_core(h_ref, s_ref, di_ref, ws_ref, b_ref, g_ref, lb_ref)
    ho_ref[...] = ho
    p_ref[...] = jnp.dot(ho, wn_ref[...], preferred_element_type=jnp.float32)


def _post_last_body(h_ref, s_ref, di_ref, ws_ref, b_ref, g_ref, lb_ref,
                    wo_ref, bo_ref, gf_ref, lbf_ref, ho_ref, e_ref):
    ho = _post_core(h_ref, s_ref, di_ref, ws_ref, b_ref, g_ref, lb_ref)
    ho_ref[...] = ho
    y = (jnp.dot(ho, wo_ref[...], preferred_element_type=jnp.float32)
         + bo_ref[...])
    e_ref[...] = _ln(y, gf_ref[...], lbf_ref[...])


_post_specs = [
    pl.BlockSpec((_R, _D), lambda i: (i, 0)),
    pl.BlockSpec((_NC, _R, _D), lambda i: (0, i, 0)),
    pl.BlockSpec((_R, 1), lambda i: (i, 0)),
    pl.BlockSpec((_D, _D), lambda i: (0, 0)),
]


def _post_fused(h, s2, di, ws, b, g, lb, wn):
    return pl.pallas_call(
        _post_body,
        grid=(_N // _R,),
        in_specs=_post_specs + [_vec(None)] * 3
        + [pl.BlockSpec((_D, _D), lambda i: (0, 0))],
        out_specs=[pl.BlockSpec((_R, _D), lambda i: (i, 0))] * 2,
        out_shape=[jax.ShapeDtypeStruct((_N, _D), jnp.float32)] * 2,
    )(h, s2, di, ws, b.reshape(1, _D), g.reshape(1, _D), lb.reshape(1, _D),
      wn)


def _post_last(h, s2, di, ws, b, g, lb, wo, bo, gf, lbf):
    return pl.pallas_call(
        _post_last_body,
        grid=(_N // _R,),
        in_specs=_post_specs + [_vec(None)] * 3
        + [pl.BlockSpec((_D, _D), lambda i: (0, 0))] + [_vec(None)] * 3,
        out_specs=[pl.BlockSpec((_R, _D), lambda i: (i, 0))] * 2,
        out_shape=[jax.ShapeDtypeStruct((_N, _D), jnp.float32)] * 2,
    )(h, s2, di, ws, b.reshape(1, _D), g.reshape(1, _D), lb.reshape(1, _D),
      wo, bo.reshape(1, _D), gf.reshape(1, _D), lbf.reshape(1, _D))


def kernel(x, edge_index, edge_weight, W_in, b_in,
           W_self_0, W_neigh_0, b_0, ln_g_0, ln_b_0,
           W_self_1, W_neigh_1, b_1, ln_g_1, ln_b_1,
           W_self_2, W_neigh_2, b_2, ln_g_2, ln_b_2,
           W_out, b_out, ln_g_f, ln_b_f):
    src = edge_index[0].astype(jnp.int32)
    dst = edge_index[1].astype(jnp.int32)
    w = edge_weight.astype(jnp.float32)
    wbits = jax.lax.bitcast_convert_type(w, jnp.int32)
    pk = jnp.stack([src.reshape(_CHUNKS, _K), dst.reshape(_CHUNKS, _K),
                    wbits.reshape(_CHUNKS, _K)], axis=1)
    z = jnp.zeros((_RPT, _D), jnp.float32)

    Ws = [W_self_0, W_self_1, W_self_2]
    Wn = [W_neigh_0, W_neigh_1, W_neigh_2]
    bs = [b_0, b_1, b_2]
    lg = [ln_g_0, ln_g_1, ln_g_2]
    lb = [ln_b_0, ln_b_1, ln_b_2]

    den2 = _sc_den(pk)
    h, p, di = _in_fused(x, W_in, b_in, Wn[0], den2)
    layer_outputs = []
    for i in range(3):
        s2 = _sc_scatter(p, pk, z)
        if i < 2:
            h, p = _post_fused(h, s2, di, Ws[i], bs[i], lg[i], lb[i],
                               Wn[i + 1])
        else:
            h, node_embeddings = _post_last(h, s2, di, Ws[i], bs[i], lg[i],
                                            lb[i], W_out, b_out, ln_g_f,
                                            ln_b_f)
        layer_outputs.append(h)

    return node_embeddings, jnp.stack(layer_outputs)


# speedup vs baseline: 1.0156x; 1.0156x over previous
import functools

import jax
import jax.numpy as jnp
from jax import lax
from jax.experimental import pallas as pl
from jax.experimental.pallas import tpu as pltpu
from jax.experimental.pallas import tpu_sc as plsc

_N = 10000
_E = 320000
_D = 128
_K = 128
_CHUNKS = _E // _K
_NC, _NS = 2, 16
_NW = _NC * _NS
_FULL = _CHUNKS // _NW
_EXTRA = _CHUNKS % _NW
_NP = 10240
_RPT = _NP // _NS

_mesh = plsc.VectorSubcoreMesh(core_axis_name="c", subcore_axis_name="s")


_GDN = lax.GatherDimensionNumbers(
    offset_dims=(), collapsed_slice_dims=(0,), start_index_map=(0,))


def _bcast_lane(vec, l):
    idx = jnp.full((16, 1), l, jnp.int32)
    return lax.gather(vec, idx, _GDN, (1,),
                      mode=lax.GatherScatterMode.PROMISE_IN_BOUNDS)


def _scale_rows(rows_v, pk_v):
    def scale_group(g, c2):
        wgrp = plsc.bitcast(pk_v[2, pl.ds(g * 16, 16)], jnp.float32)
        for l in range(16):
            wv = _bcast_lane(wgrp, l)
            e = g * 16 + l
            for q in range(_D // 16):
                sl = pl.ds(q * 16, 16)
                rows_v[e, sl] = rows_v[e, sl] * wv
        return c2

    lax.fori_loop(0, _K // 16, scale_group, 0)


def _sc_scatter_body(p_hbm, pk_hbm, z_hbm, out_hbm, *bufs):
    (pk0, pk1, pk2, rows0, rows1,
     i0, i1, i2, g0, g1, s0, s1, acc_sh) = bufs
    PK = [(pk0, i0), (pk1, i1), (pk2, i2)]
    RW = [(rows0, g0, s0), (rows1, g1, s1)]
    cid = lax.axis_index("c")
    sid = lax.axis_index("s")
    wid = sid * _NC + cid
    pltpu.sync_copy(z_hbm, acc_sh.at[pl.ds(sid * _RPT, _RPT)])
    plsc.subcore_barrier()

    def cix(j):
        return wid + j * _NW

    def issue_idx(c, k):
        pk_v, sem = PK[k]
        pltpu.async_copy(pk_hbm.at[c], pk_v, sem)

    def wait_idx(k):
        pk_v, sem = PK[k]
        pltpu.make_async_copy(pk_hbm.at[0], pk_v, sem).wait()

    def issue_gather(k, r):
        pltpu.async_copy(p_hbm.at[PK[k][0].at[0]], RW[r][0], RW[r][1])

    def wait_gather(k, r):
        pltpu.make_async_copy(p_hbm.at[PK[k][0].at[0]], RW[r][0],
                              RW[r][1]).wait()

    def issue_scatter(k, r):
        pltpu.async_copy(RW[r][0], acc_sh.at[PK[k][0].at[1]], RW[r][2],
                         add=True)

    def wait_scatter(k, r):
        pltpu.make_async_copy(RW[r][0], acc_sh.at[PK[k][0].at[1]],
                              RW[r][2]).wait()

    issue_idx(cix(0), 0)
    issue_idx(cix(1), 1)
    wait_idx(0)
    issue_gather(0, 0)

    last = _FULL - 1

    def step(t, off, j):
        wait_gather(off % 3, off % 2)

        if off == 0:
            @pl.when(t > 0)
            def _():
                wait_scatter((off + 2) % 3, (off + 1) % 2)
        else:
            wait_scatter((off + 2) % 3, (off + 1) % 2)

        def launch_next():
            wait_idx((off + 1) % 3)
            issue_gather((off + 1) % 3, (off + 1) % 2)

        def prefetch_idx():
            issue_idx(cix(j + 2), (off + 2) % 3)

        if off <= 3:
            launch_next()
            prefetch_idx()
        else:
            @pl.when(j < last)
            def _():
                launch_next()

            @pl.when(j + 2 <= last)
            def _():
                prefetch_idx()

        _scale_rows(RW[off % 2][0], PK[off % 3][0])
        issue_scatter(off % 3, off % 2)

    def six(t, carry):
        for off in range(6):
            step(t, off, 6 * t + off)
        return carry

    lax.fori_loop(0, _FULL // 6, six, 0)
    wait_scatter(2, 1)

    @pl.when(wid < _EXTRA)
    def _():
        c = _NW * _FULL + wid
        issue_idx(c, 0)
        wait_idx(0)
        issue_gather(0, 0)
        wait_gather(0, 0)
        _scale_rows(rows0, pk0)
        issue_scatter(0, 0)
        wait_scatter(0, 0)

    plsc.subcore_barrier()
    pltpu.sync_copy(acc_sh.at[pl.ds(sid * _RPT, _RPT)],
                    out_hbm.at[cid, pl.ds(sid * _RPT, _RPT)])


def _sc_bufs():
    return [
        pltpu.VMEM((3, _K), jnp.int32),
        pltpu.VMEM((3, _K), jnp.int32),
        pltpu.VMEM((3, _K), jnp.int32),
        pltpu.VMEM((_K, _D), jnp.float32),
        pltpu.VMEM((_K, _D), jnp.float32),
    ] + [pltpu.SemaphoreType.DMA] * 7


_sc_scatter = pl.kernel(
    _sc_scatter_body,
    out_type=jax.ShapeDtypeStruct((_NC, _NP, _D), jnp.float32),
    mesh=_mesh,
    scratch_types=_sc_bufs() + [pltpu.VMEM_SHARED((_NP, _D), jnp.float32)],
    compiler_params=pltpu.CompilerParams(needs_layout_passes=False),
)


def _sc_den_body(pk_hbm, out_hbm, *bufs):
    (pk0, pk1, den_v, pb, bc, i0, i1, part_sh) = bufs
    PK = [(pk0, i0), (pk1, i1)]
    cid = lax.axis_index("c")
    sid = lax.axis_index("s")
    wid = sid * _NC + cid

    def z(i, c):
        den_v[pl.ds(i * 16, 16)] = jnp.zeros((16,), jnp.float32)
        return c

    lax.fori_loop(0, _NP // 16, z, 0, unroll=8)

    def cix(j):
        return wid + j * _NW

    def issue_idx(c, k):
        pk_v, sem = PK[k]
        pltpu.async_copy(pk_hbm.at[c], pk_v, sem)

    def wait_idx(k):
        pk_v, sem = PK[k]
        pltpu.make_async_copy(pk_hbm.at[0], pk_v, sem).wait()

    def accum(k):
        pk_v, _ = PK[k]
        for g in range(_K // 16):
            sl = pl.ds(g * 16, 16)
            ii = pk_v[1, sl]
            vv = plsc.bitcast(pk_v[2, sl], jnp.float32)
            plsc.addupdate_scatter(den_v, [ii], vv)

    issue_idx(cix(0), 0)
    issue_idx(cix(1), 1)

    def two(t, carry):
        for off in range(2):
            j = 2 * t + off
            wait_idx(off)
            accum(off)

            @pl.when(j + 2 <= _FULL - 1)
            def _():
                issue_idx(cix(j + 2), off)
        return carry

    lax.fori_loop(0, _FULL // 2, two, 0)

    @pl.when(wid < _EXTRA)
    def _():
        issue_idx(_NW * _FULL + wid, 0)
        wait_idx(0)
        accum(0)

    pltpu.sync_copy(den_v, part_sh.at[sid])
    plsc.subcore_barrier()
    pltpu.sync_copy(part_sh.at[:, pl.ds(sid * _RPT, _RPT)], pb)

    def red(g, c):
        acc = pb[0, pl.ds(g * 16, 16)]
        for r in range(1, _NS):
            acc = acc + pb[r, pl.ds(g * 16, 16)]
        den_v[pl.ds(g * 16, 16)] = acc
        return c

    lax.fori_loop(0, _RPT // 16, red, 0, unroll=2)

    for b5 in range(_RPT // _K):
        def bcast(rr, c):
            wv = plsc.load_gather(
                den_v, [jnp.zeros((16,), jnp.int32) + (b5 * _K + rr)])
            for q in range(_D // 16):
                bc[rr, pl.ds(q * 16, 16)] = wv
            return c

        lax.fori_loop(0, _K, bcast, 0, unroll=4)
        pltpu.sync_copy(
            bc, out_hbm.at[cid, pl.ds(sid * _RPT + b5 * _K, _K)])


_sc_den = pl.kernel(
    _sc_den_body,
    out_type=jax.ShapeDtypeStruct((_NC, _NP, _D), jnp.float32),
    mesh=_mesh,
    scratch_types=[
        pltpu.VMEM((3, _K), jnp.int32),
        pltpu.VMEM((3, _K), jnp.int32),
        pltpu.VMEM((_NP,), jnp.float32),
        pltpu.VMEM((_NS, _RPT), jnp.float32),
        pltpu.VMEM((_K, _D), jnp.float32),
        pltpu.SemaphoreType.DMA,
        pltpu.SemaphoreType.DMA,
        pltpu.VMEM_SHARED((_NS, _NP), jnp.float32),
    ],
    compiler_params=pltpu.CompilerParams(needs_layout_passes=False),
)


_R = 1000


def _vec(b):
    return pl.BlockSpec((1, _D), lambda i: (0, 0))


def _ln(x, g, b):
    m = jnp.mean(x, axis=-1, keepdims=True)
    v = jnp.mean((x - m) * (x - m), axis=-1, keepdims=True)
    return (x - m) / jnp.sqrt(v + 1e-5) * g + b


def _dinv_body(den_ref, o_ref):
    den = den_ref[0, :, 0:1] + den_ref[1, :, 0:1]
    o_ref[...] = 1.0 / jnp.maximum(den, 1e-6)


def _dinv(den2):
    blk = _NP // 8
    return pl.pallas_call(
        _dinv_body,
        grid=(8,),
        in_specs=[pl.BlockSpec((_NC, blk, _D), lambda i: (0, i, 0))],
        out_specs=pl.BlockSpec((blk, 1), lambda i: (i, 0)),
        out_shape=jax.ShapeDtypeStruct((_NP, 1), jnp.float32),
    )(den2)


def _in_body(x_ref, wi_ref, b_ref, wn_ref, h_ref, p_ref):
    h = (jnp.dot(x_ref[...], wi_ref[...],
                 preferred_element_type=jnp.float32) + b_ref[...])
    h_ref[...] = h
    p_ref[...] = jnp.dot(h, wn_ref[...], preferred_element_type=jnp.float32)


def _in_fused(x, wi, b, wn):
    return pl.pallas_call(
        _in_body,
        grid=(_N // _R,),
        in_specs=[
            pl.BlockSpec((_R, _D), lambda i: (i, 0)),
            pl.BlockSpec((_D, _D), lambda i: (0, 0)),
            _vec(b),
            pl.BlockSpec((_D, _D), lambda i: (0, 0)),
        ],
        out_specs=[pl.BlockSpec((_R, _D), lambda i: (i, 0))] * 2,
        out_shape=[jax.ShapeDtypeStruct((_N, _D), jnp.float32)] * 2,
    )(x, wi, b.reshape(1, _D), wn)


def _post_core(h_ref, s_ref, di_ref, ws_ref, b_ref, g_ref, lb_ref):
    h = h_ref[...]
    s = s_ref[0] + s_ref[1]
    hn = (jnp.dot(h, ws_ref[...], preferred_element_type=jnp.float32)
          + s * di_ref[...] + b_ref[...])
    hn = jnp.maximum(hn, 0.0)
    hn = _ln(hn, g_ref[...], lb_ref[...])
    return h + hn


def _post_body(h_ref, s_ref, di_ref, ws_ref, b_ref, g_ref, lb_ref, wn_ref,
               ho_ref, p_ref):
    ho = _post_core(h_ref, s_ref, di_ref, ws_ref, b_ref, g_ref, lb_ref)
    ho_ref[...] = ho
    p_ref[...] = jnp.dot(ho, wn_ref[...], preferred_element_type=jnp.float32)


def _post_last_body(h_ref, s_ref, di_ref, ws_ref, b_ref, g_ref, lb_ref,
                    wo_ref, bo_ref, gf_ref, lbf_ref, ho_ref, e_ref):
    ho = _post_core(h_ref, s_ref, di_ref, ws_ref, b_ref, g_ref, lb_ref)
    ho_ref[...] = ho
    y = (jnp.dot(ho, wo_ref[...], preferred_element_type=jnp.float32)
         + bo_ref[...])
    e_ref[...] = _ln(y, gf_ref[...], lbf_ref[...])


_post_specs = [
    pl.BlockSpec((_R, _D), lambda i: (i, 0)),
    pl.BlockSpec((_NC, _R, _D), lambda i: (0, i, 0)),
    pl.BlockSpec((_R, 1), lambda i: (i, 0)),
    pl.BlockSpec((_D, _D), lambda i: (0, 0)),
]


def _post_fused(h, s2, di, ws, b, g, lb, wn):
    return pl.pallas_call(
        _post_body,
        grid=(_N // _R,),
        in_specs=_post_specs + [_vec(None)] * 3
        + [pl.BlockSpec((_D, _D), lambda i: (0, 0))],
        out_specs=[pl.BlockSpec((_R, _D), lambda i: (i, 0))] * 2,
        out_shape=[jax.ShapeDtypeStruct((_N, _D), jnp.float32)] * 2,
    )(h, s2, di, ws, b.reshape(1, _D), g.reshape(1, _D), lb.reshape(1, _D),
      wn)


def _post_last(h, s2, di, ws, b, g, lb, wo, bo, gf, lbf):
    return pl.pallas_call(
        _post_last_body,
        grid=(_N // _R,),
        in_specs=_post_specs + [_vec(None)] * 3
        + [pl.BlockSpec((_D, _D), lambda i: (0, 0))] + [_vec(None)] * 3,
        out_specs=[pl.BlockSpec((_R, _D), lambda i: (i, 0))] * 2,
        out_shape=[jax.ShapeDtypeStruct((_N, _D), jnp.float32)] * 2,
    )(h, s2, di, ws, b.reshape(1, _D), g.reshape(1, _D), lb.reshape(1, _D),
      wo, bo.reshape(1, _D), gf.reshape(1, _D), lbf.reshape(1, _D))


def kernel(x, edge_index, edge_weight, W_in, b_in,
           W_self_0, W_neigh_0, b_0, ln_g_0, ln_b_0,
           W_self_1, W_neigh_1, b_1, ln_g_1, ln_b_1,
           W_self_2, W_neigh_2, b_2, ln_g_2, ln_b_2,
           W_out, b_out, ln_g_f, ln_b_f):
    src = edge_index[0].astype(jnp.int32)
    dst = edge_index[1].astype(jnp.int32)
    w = edge_weight.astype(jnp.float32)
    wbits = jax.lax.bitcast_convert_type(w, jnp.int32)
    pk = jnp.stack([src.reshape(_CHUNKS, _K), dst.reshape(_CHUNKS, _K),
                    wbits.reshape(_CHUNKS, _K)], axis=1)
    z = jnp.zeros((_RPT, _D), jnp.float32)

    Ws = [W_self_0, W_self_1, W_self_2]
    Wn = [W_neigh_0, W_neigh_1, W_neigh_2]
    bs = [b_0, b_1, b_2]
    lg = [ln_g_0, ln_g_1, ln_g_2]
    lb = [ln_b_0, ln_b_1, ln_b_2]

    di = _dinv(_sc_den(pk))
    h, p = _in_fused(x, W_in, b_in, Wn[0])
    layer_outputs = []
    for i in range(3):
        s2 = _sc_scatter(p, pk, z)
        if i < 2:
            h, p = _post_fused(h, s2, di, Ws[i], bs[i], lg[i], lb[i],
                               Wn[i + 1])
        else:
            h, node_embeddings = _post_last(h, s2, di, Ws[i], bs[i], lg[i],
                                            lb[i], W_out, b_out, ln_g_f,
                                            ln_b_f)
        layer_outputs.append(h)

    return node_embeddings, jnp.stack(layer_outputs)
